# baseline (speedup 1.0000x reference)
import functools

import jax
import jax.numpy as jnp
from jax.experimental import pallas as pl
from jax.experimental.pallas import tpu as pltpu

S, D, E, TOPK, FF = 2048, 768, 8, 2, 2048
FB = 256
NFB = FF // FB


def _gating_body(x_ref, wr_ref, combine_ref, loss_ref, xbf_ref):
    x = x_ref[...]
    wr = wr_ref[...]
    xbf_ref[...] = x.astype(jnp.bfloat16)
    logits = jax.lax.dot_general(
        x, wr, (((1,), (1,)), ((), ())),
        preferred_element_type=jnp.float32,
    )
    m = jnp.max(logits, axis=-1, keepdims=True)
    p = jnp.exp(logits - m)
    p = p / jnp.sum(p, axis=-1, keepdims=True)

    g1 = jnp.max(p, axis=-1, keepdims=True)
    i1 = jnp.argmax(p, axis=-1, keepdims=True)
    lanes = jax.lax.broadcasted_iota(jnp.int32, (S, E), 1)
    p2 = jnp.where(lanes == i1, -jnp.inf, p)
    g2 = jnp.max(p2, axis=-1, keepdims=True)
    i2 = jnp.argmax(p2, axis=-1, keepdims=True)
    denom = g1 + g2 + 1e-8
    w1 = g1 / denom
    w2 = g2 / denom

    sel1 = (lanes == i1).astype(jnp.float32)
    sel2 = (lanes == i2).astype(jnp.float32)
    combine_ref[...] = sel1 * w1 + sel2 * w2

    counts = jnp.sum(sel1 + sel2, axis=0)
    usage = counts / jnp.sum(counts)
    mean = jnp.mean(usage)
    var = jnp.sum((usage - mean) ** 2) / (E - 1)
    cv2 = (var / (mean + 1e-8)) ** 2
    loss_ref[0, 0] = cv2


def _ffn_body(x_ref, wg_ref, wu_ref, wd_ref, combine_ref, out_ref):
    e = pl.program_id(0)
    f = pl.program_id(1)

    @pl.when(jnp.logical_and(e == 0, f == 0))
    def _init():
        out_ref[...] = jnp.zeros_like(out_ref)

    x = x_ref[...]
    wg = wg_ref[0].astype(jnp.bfloat16)
    wu = wu_ref[0].astype(jnp.bfloat16)
    wd = wd_ref[0, :, pl.ds(f * FB, FB)].astype(jnp.bfloat16)
    g = jax.lax.dot_general(
        x, wg, (((1,), (1,)), ((), ())),
        preferred_element_type=jnp.float32)
    u = jax.lax.dot_general(
        x, wu, (((1,), (1,)), ((), ())),
        preferred_element_type=jnp.float32)
    act = (g * (u * jax.nn.sigmoid(u))).astype(jnp.bfloat16)
    part = jax.lax.dot_general(
        act, wd, (((1,), (1,)), ((), ())),
        preferred_element_type=jnp.float32)

    lanes = jax.lax.broadcasted_iota(jnp.int32, (S, E), 1)
    c = combine_ref[...]
    wcol = jnp.sum(jnp.where(lanes == e, c, 0.0), axis=1, keepdims=True)
    out_ref[...] += wcol * part


@jax.jit
def kernel(x, Wg, Wu, Wd, Wr):
    b, s, d = x.shape
    x2 = x.reshape(s, d)

    combine, loss, x_bf = pl.pallas_call(
        _gating_body,
        out_shape=(
            jax.ShapeDtypeStruct((S, E), jnp.float32),
            jax.ShapeDtypeStruct((1, 1), jnp.float32),
            jax.ShapeDtypeStruct((S, D), jnp.bfloat16),
        ),
        in_specs=[
            pl.BlockSpec((S, D), lambda: (0, 0)),
            pl.BlockSpec((E, D), lambda: (0, 0)),
        ],
        out_specs=(
            pl.BlockSpec((S, E), lambda: (0, 0)),
            pl.BlockSpec(memory_space=pltpu.SMEM),
            pl.BlockSpec((S, D), lambda: (0, 0)),
        ),
    )(x2, Wr)

    out = pl.pallas_call(
        _ffn_body,
        grid=(E, NFB),
        out_shape=jax.ShapeDtypeStruct((S, D), jnp.float32),
        in_specs=[
            pl.BlockSpec((S, D), lambda e, f: (0, 0)),
            pl.BlockSpec((1, FB, D), lambda e, f: (e, f, 0)),
            pl.BlockSpec((1, FB, D), lambda e, f: (e, f, 0)),
            pl.BlockSpec((1, D, FF), lambda e, f: (e, 0, 0)),
            pl.BlockSpec((S, E), lambda e, f: (0, 0)),
        ],
        out_specs=pl.BlockSpec((S, D), lambda e, f: (0, 0)),
    )(x_bf, Wg, Wu, Wd, combine)

    return out.reshape(b, s, d), loss.reshape(())


# speedup vs baseline: 1.1143x; 1.1143x over previous
import functools

import jax
import jax.numpy as jnp
from jax.experimental import pallas as pl
from jax.experimental.pallas import tpu as pltpu

S, D, E, TOPK, FF = 2048, 768, 8, 2, 2048
FB = 512
NFB = FF // FB


def _gating_body(x_ref, wr_ref, combine_ref, loss_ref, xbf_ref):
    x = x_ref[...]
    wr = wr_ref[...]
    xbf_ref[...] = x.astype(jnp.bfloat16)
    logits = jax.lax.dot_general(
        x, wr, (((1,), (1,)), ((), ())),
        preferred_element_type=jnp.float32,
    )
    m = jnp.max(logits, axis=-1, keepdims=True)
    p = jnp.exp(logits - m)
    p = p / jnp.sum(p, axis=-1, keepdims=True)

    g1 = jnp.max(p, axis=-1, keepdims=True)
    i1 = jnp.argmax(p, axis=-1, keepdims=True)
    lanes = jax.lax.broadcasted_iota(jnp.int32, (S, E), 1)
    p2 = jnp.where(lanes == i1, -jnp.inf, p)
    g2 = jnp.max(p2, axis=-1, keepdims=True)
    i2 = jnp.argmax(p2, axis=-1, keepdims=True)
    denom = g1 + g2 + 1e-8
    w1 = g1 / denom
    w2 = g2 / denom

    sel1 = (lanes == i1).astype(jnp.float32)
    sel2 = (lanes == i2).astype(jnp.float32)
    combine_ref[...] = sel1 * w1 + sel2 * w2

    counts = jnp.sum(sel1 + sel2, axis=0)
    usage = counts / jnp.sum(counts)
    mean = jnp.mean(usage)
    var = jnp.sum((usage - mean) ** 2) / (E - 1)
    cv2 = (var / (mean + 1e-8)) ** 2
    loss_ref[0, 0] = cv2


def _ffn_body(x_ref, wg_ref, wu_ref, wd_ref, combine_ref, out_ref):
    e = pl.program_id(0)
    f = pl.program_id(1)

    @pl.when(jnp.logical_and(e == 0, f == 0))
    def _init():
        out_ref[...] = jnp.zeros_like(out_ref)

    x = x_ref[...]
    wg = wg_ref[0].astype(jnp.bfloat16)
    wu = wu_ref[0].astype(jnp.bfloat16)
    wd = wd_ref[0, :, pl.ds(f * FB, FB)].astype(jnp.bfloat16)
    g = jax.lax.dot_general(
        x, wg, (((1,), (1,)), ((), ())),
        preferred_element_type=jnp.float32)
    u = jax.lax.dot_general(
        x, wu, (((1,), (1,)), ((), ())),
        preferred_element_type=jnp.float32)
    act = (g * (u * jax.nn.sigmoid(u))).astype(jnp.bfloat16)
    part = jax.lax.dot_general(
        act, wd, (((1,), (1,)), ((), ())),
        preferred_element_type=jnp.float32)

    lanes = jax.lax.broadcasted_iota(jnp.int32, (S, E), 1)
    c = combine_ref[...]
    wcol = jnp.sum(jnp.where(lanes == e, c, 0.0), axis=1, keepdims=True)
    out_ref[...] += wcol * part


@jax.jit
def kernel(x, Wg, Wu, Wd, Wr):
    b, s, d = x.shape
    x2 = x.reshape(s, d)

    combine, loss, x_bf = pl.pallas_call(
        _gating_body,
        out_shape=(
            jax.ShapeDtypeStruct((S, E), jnp.float32),
            jax.ShapeDtypeStruct((1, 1), jnp.float32),
            jax.ShapeDtypeStruct((S, D), jnp.bfloat16),
        ),
        in_specs=[
            pl.BlockSpec((S, D), lambda: (0, 0)),
            pl.BlockSpec((E, D), lambda: (0, 0)),
        ],
        out_specs=(
            pl.BlockSpec((S, E), lambda: (0, 0)),
            pl.BlockSpec(memory_space=pltpu.SMEM),
            pl.BlockSpec((S, D), lambda: (0, 0)),
        ),
    )(x2, Wr)

    out = pl.pallas_call(
        _ffn_body,
        grid=(E, NFB),
        out_shape=jax.ShapeDtypeStruct((S, D), jnp.float32),
        in_specs=[
            pl.BlockSpec((S, D), lambda e, f: (0, 0)),
            pl.BlockSpec((1, FB, D), lambda e, f: (e, f, 0)),
            pl.BlockSpec((1, FB, D), lambda e, f: (e, f, 0)),
            pl.BlockSpec((1, D, FF), lambda e, f: (e, 0, 0)),
            pl.BlockSpec((S, E), lambda e, f: (0, 0)),
        ],
        out_specs=pl.BlockSpec((S, D), lambda e, f: (0, 0)),
    )(x_bf, Wg, Wu, Wd, combine)

    return out.reshape(b, s, d), loss.reshape(())


# speedup vs baseline: 1.1661x; 1.0465x over previous
import functools

import jax
import jax.numpy as jnp
from jax.experimental import pallas as pl
from jax.experimental.pallas import tpu as pltpu

S, D, E, TOPK, FF = 2048, 768, 8, 2, 2048
FB = 1024
NFB = FF // FB


def _gating_body(x_ref, wr_ref, combine_ref, loss_ref, xbf_ref):
    x = x_ref[...]
    wr = wr_ref[...]
    xbf_ref[...] = x.astype(jnp.bfloat16)
    logits = jax.lax.dot_general(
        x, wr, (((1,), (1,)), ((), ())),
        preferred_element_type=jnp.float32,
    )
    m = jnp.max(logits, axis=-1, keepdims=True)
    p = jnp.exp(logits - m)
    p = p / jnp.sum(p, axis=-1, keepdims=True)

    g1 = jnp.max(p, axis=-1, keepdims=True)
    i1 = jnp.argmax(p, axis=-1, keepdims=True)
    lanes = jax.lax.broadcasted_iota(jnp.int32, (S, E), 1)
    p2 = jnp.where(lanes == i1, -jnp.inf, p)
    g2 = jnp.max(p2, axis=-1, keepdims=True)
    i2 = jnp.argmax(p2, axis=-1, keepdims=True)
    denom = g1 + g2 + 1e-8
    w1 = g1 / denom
    w2 = g2 / denom

    sel1 = (lanes == i1).astype(jnp.float32)
    sel2 = (lanes == i2).astype(jnp.float32)
    combine_ref[...] = sel1 * w1 + sel2 * w2

    counts = jnp.sum(sel1 + sel2, axis=0)
    usage = counts / jnp.sum(counts)
    mean = jnp.mean(usage)
    var = jnp.sum((usage - mean) ** 2) / (E - 1)
    cv2 = (var / (mean + 1e-8)) ** 2
    loss_ref[0, 0] = cv2


def _ffn_body(x_ref, wg_ref, wu_ref, wd_ref, combine_ref, out_ref):
    e = pl.program_id(0)
    f = pl.program_id(1)

    @pl.when(jnp.logical_and(e == 0, f == 0))
    def _init():
        out_ref[...] = jnp.zeros_like(out_ref)

    x = x_ref[...]
    wg = wg_ref[0].astype(jnp.bfloat16)
    wu = wu_ref[0].astype(jnp.bfloat16)
    wd = wd_ref[0, :, pl.ds(f * FB, FB)].astype(jnp.bfloat16)
    g = jax.lax.dot_general(
        x, wg, (((1,), (1,)), ((), ())),
        preferred_element_type=jnp.float32)
    u = jax.lax.dot_general(
        x, wu, (((1,), (1,)), ((), ())),
        preferred_element_type=jnp.float32)
    act = (g * (u * jax.nn.sigmoid(u))).astype(jnp.bfloat16)
    part = jax.lax.dot_general(
        act, wd, (((1,), (1,)), ((), ())),
        preferred_element_type=jnp.float32)

    lanes = jax.lax.broadcasted_iota(jnp.int32, (S, E), 1)
    c = combine_ref[...]
    wcol = jnp.sum(jnp.where(lanes == e, c, 0.0), axis=1, keepdims=True)
    out_ref[...] += wcol * part


@jax.jit
def kernel(x, Wg, Wu, Wd, Wr):
    b, s, d = x.shape
    x2 = x.reshape(s, d)

    combine, loss, x_bf = pl.pallas_call(
        _gating_body,
        out_shape=(
            jax.ShapeDtypeStruct((S, E), jnp.float32),
            jax.ShapeDtypeStruct((1, 1), jnp.float32),
            jax.ShapeDtypeStruct((S, D), jnp.bfloat16),
        ),
        in_specs=[
            pl.BlockSpec((S, D), lambda: (0, 0)),
            pl.BlockSpec((E, D), lambda: (0, 0)),
        ],
        out_specs=(
            pl.BlockSpec((S, E), lambda: (0, 0)),
            pl.BlockSpec(memory_space=pltpu.SMEM),
            pl.BlockSpec((S, D), lambda: (0, 0)),
        ),
    )(x2, Wr)

    out = pl.pallas_call(
        _ffn_body,
        grid=(E, NFB),
        out_shape=jax.ShapeDtypeStruct((S, D), jnp.float32),
        in_specs=[
            pl.BlockSpec((S, D), lambda e, f: (0, 0)),
            pl.BlockSpec((1, FB, D), lambda e, f: (e, f, 0)),
            pl.BlockSpec((1, FB, D), lambda e, f: (e, f, 0)),
            pl.BlockSpec((1, D, FF), lambda e, f: (e, 0, 0)),
            pl.BlockSpec((S, E), lambda e, f: (0, 0)),
        ],
        out_specs=pl.BlockSpec((S, D), lambda e, f: (0, 0)),
    )(x_bf, Wg, Wu, Wd, combine)

    return out.reshape(b, s, d), loss.reshape(())
